# baseline (speedup 1.0000x reference)
import jax
import jax.numpy as jnp
from jax import lax
from jax.experimental import pallas as pl
from jax.experimental.pallas import tpu as pltpu

_M = 32
_N = 32
_MAX_EPOCHS = 100
_INITIAL_RADIUS = max(_M, _N) / 2.0
_INITIAL_LR = 0.1
_STD_COEFF = 0.5

_KEY_BIAS = 1024.0
_KEY_CLIP = 2040.0
_KEY_SCALE = 512.0
_IDX_BITS = 10


def _som_body(epoch_ref, x_ref, w_ref, loc_ref, out_w_ref, stage_ref, sem_ref):
    x = x_ref[...]
    w = w_ref[...]

    B = x.shape[0]
    K = w.shape[0]

    epoch_f = epoch_ref[0].astype(jnp.float32)
    radius = _INITIAL_RADIUS - epoch_f * (
        (_INITIAL_RADIUS - 1.0) / float(_MAX_EPOCHS - 1)
    )
    alpha = _INITIAL_LR * (1.0 - epoch_f / float(_MAX_EPOCHS))
    neg_inv_two_sigma2 = -1.0 / (2.0 * (radius * _STD_COEFF) ** 2)

    wT = w.T
    wn = jnp.sum(wT * wT, axis=0)

    xh = x.astype(jnp.bfloat16)
    xl = (x - xh.astype(jnp.float32)) * (-2.0 * _KEY_SCALE)
    xhs = (xh.astype(jnp.float32) * (-2.0 * _KEY_SCALE)).astype(jnp.bfloat16)
    xls = xl.astype(jnp.bfloat16)
    wTh = wT.astype(jnp.bfloat16)
    wTl = (wT - wTh.astype(jnp.float32)).astype(jnp.bfloat16)

    def _dot(a, b):
        return lax.dot_general(
            a, b, (((1,), (0,)), ((), ())),
            preferred_element_type=jnp.float32,
        )

    xw2s = _dot(xhs, wTh) + (_dot(xhs, wTl) + _dot(xls, wTh))
    wnb = (wn + _KEY_BIAS) * _KEY_SCALE
    dq = jnp.clip(wnb[None, :] + xw2s, 0.0, _KEY_CLIP * _KEY_SCALE)

    kk = lax.broadcasted_iota(jnp.int32, (B, K), 1)
    key = (dq.astype(jnp.int32) << _IDX_BITS) | kk

    def _lane_min(a):
        c = a[:, 0:128]
        for t in range(1, K // 128):
            c = jnp.minimum(c, a[:, 128 * t:128 * (t + 1)])
        return jnp.min(c, axis=1, keepdims=True)

    m1 = _lane_min(key)
    key2 = jnp.where(key == m1, jnp.int32(2**30), key)
    m2 = _lane_min(key2)
    i1 = (m1 & (2**_IDX_BITS - 1))[:, 0]
    i2 = (m2 & (2**_IDX_BITS - 1))[:, 0]

    oh1 = (key == m1).astype(jnp.bfloat16)
    oh2 = (key2 == m2).astype(jnp.bfloat16)
    w1 = w.astype(jnp.bfloat16)
    r1 = w - w1.astype(jnp.float32)
    w2 = r1.astype(jnp.bfloat16)
    w3 = (r1 - w2.astype(jnp.float32)).astype(jnp.bfloat16)
    g1 = _dot(oh1, w1) + (_dot(oh1, w2) + _dot(oh1, w3))
    g2 = _dot(oh2, w1) + (_dot(oh2, w2) + _dot(oh2, w3))
    e1 = jnp.sum((g1 - x) ** 2, axis=1)
    e2 = jnp.sum((g2 - x) ** 2, axis=1)
    take2 = (e2 < e1) | ((e2 == e1) & (i2 < i1))
    bmu = jnp.where(take2, i2, i1)

    bi = bmu // _N
    bj = bmu - bi * _N
    ui = lax.broadcasted_iota(jnp.int32, (_M, B), 0)
    fa = alpha * jnp.exp(
        ((ui - bi[None, :]) ** 2).astype(jnp.float32) * neg_inv_two_sigma2
    )
    fb = jnp.exp(
        ((ui - bj[None, :]) ** 2).astype(jnp.float32) * neg_inv_two_sigma2
    )
    lrT = (fa[:, None, :] * fb[None, :, :]).reshape(K, B)

    loc_ref[...] = jnp.concatenate([bi[None, :], bj[None, :]], axis=0)
    KH = K // 2
    copies = []
    for h in range(2):
        lrTh = lrT[h * KH:(h + 1) * KH]
        numh = _dot(lrTh.astype(jnp.bfloat16), xh)
        denh = jnp.sum(lrTh, axis=1) + 1e-12
        stage_ref[h] = numh / denh[:, None]
        cp = pltpu.make_async_copy(
            stage_ref.at[h],
            out_w_ref.at[pl.ds(h * KH, KH), :],
            sem_ref.at[h],
        )
        cp.start()
        copies.append(cp)
    for cp in copies:
        cp.wait()


def kernel(input_vect, weights, epoch):
    B, D = input_vect.shape
    K = weights.shape[0]

    epoch_arr = jnp.asarray(epoch, jnp.int32).reshape((1,))

    loc2, new_weights = pl.pallas_call(
        _som_body,
        out_shape=(
            jax.ShapeDtypeStruct((2, B), jnp.int32),
            jax.ShapeDtypeStruct((K, D), jnp.float32),
        ),
        in_specs=[
            pl.BlockSpec(memory_space=pltpu.SMEM),
            pl.BlockSpec(memory_space=pltpu.VMEM),
            pl.BlockSpec(memory_space=pltpu.VMEM),
        ],
        out_specs=(
            pl.BlockSpec(memory_space=pltpu.VMEM),
            pl.BlockSpec(memory_space=pltpu.MemorySpace.HBM),
        ),
        scratch_shapes=[
            pltpu.VMEM((2, K // 2, D), jnp.float32),
            pltpu.SemaphoreType.DMA((2,)),
        ],
    )(epoch_arr, input_vect, weights)

    return loc2.T, new_weights


# speedup vs baseline: 1.1038x; 1.1038x over previous
import jax
import jax.numpy as jnp
from jax import lax
from jax.experimental import pallas as pl
from jax.experimental.pallas import tpu as pltpu

_M = 32
_N = 32
_MAX_EPOCHS = 100
_INITIAL_RADIUS = max(_M, _N) / 2.0
_INITIAL_LR = 0.1
_STD_COEFF = 0.5

_KEY_BIAS = 1024.0
_KEY_CLIP = 2040.0
_KEY_SCALE = 512.0
_IDX_BITS = 10


def _som_body(epoch_ref, x_ref, w_ref, loc_ref, out_w_ref):
    x = x_ref[...]
    w = w_ref[...]

    B = x.shape[0]
    K = w.shape[0]

    epoch_f = epoch_ref[0].astype(jnp.float32)
    radius = _INITIAL_RADIUS - epoch_f * (
        (_INITIAL_RADIUS - 1.0) / float(_MAX_EPOCHS - 1)
    )
    alpha = _INITIAL_LR * (1.0 - epoch_f / float(_MAX_EPOCHS))
    neg_inv_two_sigma2 = -1.0 / (2.0 * (radius * _STD_COEFF) ** 2)

    wT = w.T
    wn = jnp.sum(wT * wT, axis=0)

    xh = x.astype(jnp.bfloat16)
    xl = (x - xh.astype(jnp.float32)) * (-2.0 * _KEY_SCALE)
    xhs = (xh.astype(jnp.float32) * (-2.0 * _KEY_SCALE)).astype(jnp.bfloat16)
    xls = xl.astype(jnp.bfloat16)
    wTh = wT.astype(jnp.bfloat16)
    wTl = (wT - wTh.astype(jnp.float32)).astype(jnp.bfloat16)

    def _dot(a, b):
        return lax.dot_general(
            a, b, (((1,), (0,)), ((), ())),
            preferred_element_type=jnp.float32,
        )

    xw2s = _dot(xhs, wTh) + (_dot(xhs, wTl) + _dot(xls, wTh))
    wnb = (wn + _KEY_BIAS) * _KEY_SCALE
    dq = jnp.clip(wnb[None, :] + xw2s, 0.0, _KEY_CLIP * _KEY_SCALE)

    kk = lax.broadcasted_iota(jnp.int32, (B, K), 1)
    key = (dq.astype(jnp.int32) << _IDX_BITS) | kk

    def _lane_min(a):
        c = a[:, 0:128]
        for t in range(1, K // 128):
            c = jnp.minimum(c, a[:, 128 * t:128 * (t + 1)])
        return jnp.min(c, axis=1, keepdims=True)

    keyf = lax.bitcast_convert_type(key, jnp.float32)
    m1 = _lane_min(keyf)
    key2 = jnp.where(keyf == m1, jnp.float32(2.0), keyf)
    m2 = _lane_min(key2)
    m1i = lax.bitcast_convert_type(m1, jnp.int32)
    m2i = lax.bitcast_convert_type(m2, jnp.int32)
    i1 = (m1i & (2**_IDX_BITS - 1))[:, 0]
    i2 = (m2i & (2**_IDX_BITS - 1))[:, 0]

    oh1 = (keyf == m1).astype(jnp.bfloat16)
    oh2 = (key2 == m2).astype(jnp.bfloat16)
    w1 = w.astype(jnp.bfloat16)
    r1 = w - w1.astype(jnp.float32)
    w2 = r1.astype(jnp.bfloat16)
    w3 = (r1 - w2.astype(jnp.float32)).astype(jnp.bfloat16)
    g1 = _dot(oh1, w1) + (_dot(oh1, w2) + _dot(oh1, w3))
    g2 = _dot(oh2, w1) + (_dot(oh2, w2) + _dot(oh2, w3))
    e1 = jnp.sum((g1 - x) ** 2, axis=1)
    e2 = jnp.sum((g2 - x) ** 2, axis=1)
    take2 = (e2 < e1) | ((e2 == e1) & (i2 < i1))
    bmu = jnp.where(take2, i2, i1)

    bi = bmu // _N
    bj = bmu - bi * _N
    ui = lax.broadcasted_iota(jnp.int32, (_M, B), 0)
    fa = alpha * jnp.exp(
        ((ui - bi[None, :]) ** 2).astype(jnp.float32) * neg_inv_two_sigma2
    )
    fb = jnp.exp(
        ((ui - bj[None, :]) ** 2).astype(jnp.float32) * neg_inv_two_sigma2
    )
    lrT = (fa[:, None, :] * fb[None, :, :]).reshape(K, B)

    num = _dot(lrT.astype(jnp.bfloat16), xh)
    den = jnp.sum(lrT, axis=1) + 1e-12
    out_w_ref[...] = num / den[:, None]
    loc_ref[...] = jnp.concatenate([bi[None, :], bj[None, :]], axis=0)


def kernel(input_vect, weights, epoch):
    B, D = input_vect.shape
    K = weights.shape[0]

    epoch_arr = jnp.asarray(epoch, jnp.int32).reshape((1,))

    loc2, new_weights = pl.pallas_call(
        _som_body,
        out_shape=(
            jax.ShapeDtypeStruct((2, B), jnp.int32),
            jax.ShapeDtypeStruct((K, D), jnp.float32),
        ),
        in_specs=[
            pl.BlockSpec(memory_space=pltpu.SMEM),
            pl.BlockSpec(memory_space=pltpu.VMEM),
            pl.BlockSpec(memory_space=pltpu.VMEM),
        ],
        out_specs=(
            pl.BlockSpec(memory_space=pltpu.VMEM),
            pl.BlockSpec(memory_space=pltpu.VMEM),
        ),
    )(epoch_arr, input_vect, weights)

    return loc2.T, new_weights
